# baseline (speedup 1.0000x reference)
import functools

import numpy as np
import jax
import jax.numpy as jnp
from jax import lax
from jax.experimental import pallas as pl
from jax.experimental.pallas import tpu as pltpu
from jax.experimental.pallas import tpu_sc as plsc

_D = 3
_L = 16
_F = 2
_T = 19
_H = 2 ** _T
_N_MIN = 16
_N_MAX = 2048
_B = float(np.exp2(np.log2(_N_MAX * 1.0 / _N_MIN) / (_L - 1)))
_P1 = np.uint32(2654435761)
_P2 = np.uint32(805459861)

_N = 524288
_NW = 32
_PW = _N // _NW
_C = 1024
_NCH = _PW // _C
_G1 = _C // 16
_G2 = _C // 8


def _level_params():
    params = []
    for l in range(_L):
        gscale = float(np.exp2(l * np.log2(_B)) * _N_MIN - 1.0)
        res = int(np.ceil(gscale)) + 1
        params.append((gscale, res, res ** _D <= _H))
    return params


_LEVELS = _level_params()


def _sc_kernel(x0_hbm, x1_hbm, x2_hbm, table_hbm, out_hbm,
               xv0, xv1, xv2, idxr, rows, outb, sem):
    wid = lax.axis_index("c") * 16 + lax.axis_index("s")

    def chunk_body(ci, carry):
        gbase = wid * _PW + ci * _C
        pltpu.sync_copy(x0_hbm.at[pl.ds(gbase, _C)], xv0)
        pltpu.sync_copy(x1_hbm.at[pl.ds(gbase, _C)], xv1)
        pltpu.sync_copy(x2_hbm.at[pl.ds(gbase, _C)], xv2)

        for l, (gscale, res, dense) in enumerate(_LEVELS):
            gs = jnp.float32(gscale)

            def pass1(i, c1):
                o = i * 16
                p0 = (xv0[pl.ds(o, 16)] * gs + 0.5).astype(jnp.int32)
                p1 = (xv1[pl.ds(o, 16)] * gs + 0.5).astype(jnp.int32)
                p2 = (xv2[pl.ds(o, 16)] * gs + 0.5).astype(jnp.int32)
                if dense:
                    a = [p0, p0 + 1]
                    b = [p1 * res, (p1 + 1) * res]
                    c = [p2 * (res * res), (p2 + 1) * (res * res)]
                    for corner in range(8):
                        idx = (a[corner & 1] + b[(corner >> 1) & 1]
                               + c[(corner >> 2) & 1] + l * _H)
                        idxr[pl.ds(corner * _C + o, 16)] = idx
                else:
                    u0 = p0.astype(jnp.uint32)
                    h1 = p1.astype(jnp.uint32) * _P1
                    h2 = p2.astype(jnp.uint32) * _P2
                    a = [u0, u0 + np.uint32(1)]
                    b = [h1, h1 + _P1]
                    c = [h2, h2 + _P2]
                    mask = jnp.uint32(_H - 1)
                    for corner in range(8):
                        v = a[corner & 1] ^ b[(corner >> 1) & 1] ^ c[(corner >> 2) & 1]
                        idx = (v & mask).astype(jnp.int32) + l * _H
                        idxr[pl.ds(corner * _C + o, 16)] = idx
                return c1

            lax.fori_loop(0, _G1, pass1, 0)

            pltpu.async_copy(table_hbm.at[idxr], rows, sem).wait()

            def pass2(i, c2):
                p8 = i * 8
                iota = lax.iota(jnp.int32, 16)
                half = lax.shift_right_logical(iota, 1)
                feat = iota & 1
                dup = p8 + half
                x0d = plsc.load_gather(xv0, [dup])
                x1d = plsc.load_gather(xv1, [dup])
                x2d = plsc.load_gather(xv2, [dup])
                pos0 = x0d * gs + 0.5
                pos1 = x1d * gs + 0.5
                pos2 = x2d * gs + 0.5
                f0 = pos0 - pos0.astype(jnp.int32).astype(jnp.float32)
                f1 = pos1 - pos1.astype(jnp.int32).astype(jnp.float32)
                f2 = pos2 - pos2.astype(jnp.int32).astype(jnp.float32)
                t0 = 1.0 - f0
                t1 = 1.0 - f1
                t2 = 1.0 - f2
                w01 = [t0 * t1, f0 * t1, t0 * f1, f0 * f1]
                w2 = [t2, f2]
                acc = jnp.zeros((16,), jnp.float32)
                for corner in range(8):
                    vals = plsc.load_gather(rows, [dup + corner * _C, feat])
                    wc = w01[corner & 3] * w2[(corner >> 2) & 1]
                    acc = acc + wc * vals
                plsc.store_scatter(outb, [dup, 2 * l + feat], acc)
                return c2

            lax.fori_loop(0, _G2, pass2, 0)

        pltpu.sync_copy(outb, out_hbm.at[pl.ds(gbase, _C), :])
        return carry

    lax.fori_loop(0, _NCH, chunk_body, 0)


@jax.jit
def kernel(x, table):
    x0 = x[:, 0]
    x1 = x[:, 1]
    x2 = x[:, 2]
    table_flat = table.reshape(_L * _H, _F)
    mesh = plsc.VectorSubcoreMesh(core_axis_name="c", subcore_axis_name="s")
    run = functools.partial(
        pl.kernel,
        out_type=jax.ShapeDtypeStruct((_N, _L * _F), jnp.float32),
        mesh=mesh,
        scratch_types=[
            pltpu.VMEM((_C,), jnp.float32),
            pltpu.VMEM((_C,), jnp.float32),
            pltpu.VMEM((_C,), jnp.float32),
            pltpu.VMEM((8 * _C,), jnp.int32),
            pltpu.VMEM((8 * _C, _F), jnp.float32),
            pltpu.VMEM((_C, _L * _F), jnp.float32),
            pltpu.SemaphoreType.DMA,
        ],
        compiler_params=pltpu.CompilerParams(
            needs_layout_passes=False, use_tc_tiling_on_sc=False),
    )(_sc_kernel)
    return run(x0, x1, x2, table_flat)


# speedup vs baseline: 2.8576x; 2.8576x over previous
import functools

import numpy as np
import jax
import jax.numpy as jnp
from jax import lax
from jax.experimental import pallas as pl
from jax.experimental.pallas import tpu as pltpu
from jax.experimental.pallas import tpu_sc as plsc

_D = 3
_L = 16
_F = 2
_T = 19
_H = 2 ** _T
_N_MIN = 16
_N_MAX = 2048
_B = float(np.exp2(np.log2(_N_MAX * 1.0 / _N_MIN) / (_L - 1)))
_P1 = np.uint32(2654435761)
_P2 = np.uint32(805459861)

_N = 524288
_NW = 32
_PW = _N // _NW
_C = 1024
_NCH = _PW // _C
_G = _C // 16


def _level_params():
    params = []
    for l in range(_L):
        gscale = float(np.exp2(l * np.log2(_B)) * _N_MIN - 1.0)
        res = int(np.ceil(gscale)) + 1
        params.append((gscale, res, res ** _D <= _H))
    return params


_LEVELS = _level_params()


def _flat_addr(idx_i32, l):
    return (idx_i32 & 127) | ((idx_i32 >> 7) << 8) | (l << 20)


def _sc_kernel(x0_hbm, x1_hbm, x2_hbm, table_hbm, out_hbm,
               xv0, xv1, xv2, idxr, rows, outb, sem):
    wid = lax.axis_index("c") * 16 + lax.axis_index("s")

    def chunk_body(ci, carry):
        gbase = wid * _PW + ci * _C
        pltpu.sync_copy(x0_hbm.at[pl.ds(gbase, _C)], xv0)
        pltpu.sync_copy(x1_hbm.at[pl.ds(gbase, _C)], xv1)
        pltpu.sync_copy(x2_hbm.at[pl.ds(gbase, _C)], xv2)

        for l, (gscale, res, dense) in enumerate(_LEVELS):
            gs = jnp.float32(gscale)

            def pass1(i, c1):
                o = i * 16
                p0 = (xv0[pl.ds(o, 16)] * gs + 0.5).astype(jnp.int32)
                p1 = (xv1[pl.ds(o, 16)] * gs + 0.5).astype(jnp.int32)
                p2 = (xv2[pl.ds(o, 16)] * gs + 0.5).astype(jnp.int32)
                if dense:
                    a = [p0, p0 + 1]
                    b = [p1 * res, (p1 + 1) * res]
                    c = [p2 * (res * res), (p2 + 1) * (res * res)]
                    for corner in range(8):
                        idx = a[corner & 1] + b[(corner >> 1) & 1] + c[(corner >> 2) & 1]
                        a0 = _flat_addr(idx, l)
                        idxr[pl.ds(corner * _C + o, 16)] = a0
                        idxr[pl.ds(8 * _C + corner * _C + o, 16)] = a0 + 128
                else:
                    u0 = p0.astype(jnp.uint32)
                    h1 = p1.astype(jnp.uint32) * _P1
                    h2 = p2.astype(jnp.uint32) * _P2
                    a = [u0, u0 + np.uint32(1)]
                    b = [h1, h1 + _P1]
                    c = [h2, h2 + _P2]
                    mask = jnp.uint32(_H - 1)
                    for corner in range(8):
                        v = a[corner & 1] ^ b[(corner >> 1) & 1] ^ c[(corner >> 2) & 1]
                        a0 = _flat_addr((v & mask).astype(jnp.int32), l)
                        idxr[pl.ds(corner * _C + o, 16)] = a0
                        idxr[pl.ds(8 * _C + corner * _C + o, 16)] = a0 + 128
                return c1

            lax.fori_loop(0, _G, pass1, 0)

            pltpu.async_copy(table_hbm.at[idxr], rows, sem).wait()

            def pass2(i, c2):
                o = i * 16
                pos0 = xv0[pl.ds(o, 16)] * gs + 0.5
                pos1 = xv1[pl.ds(o, 16)] * gs + 0.5
                pos2 = xv2[pl.ds(o, 16)] * gs + 0.5
                f0 = pos0 - pos0.astype(jnp.int32).astype(jnp.float32)
                f1 = pos1 - pos1.astype(jnp.int32).astype(jnp.float32)
                f2 = pos2 - pos2.astype(jnp.int32).astype(jnp.float32)
                t0 = 1.0 - f0
                t1 = 1.0 - f1
                t2 = 1.0 - f2
                w01 = [t0 * t1, f0 * t1, t0 * f1, f0 * f1]
                w2 = [t2, f2]
                acc0 = jnp.zeros((16,), jnp.float32)
                acc1 = jnp.zeros((16,), jnp.float32)
                for corner in range(8):
                    wc = w01[corner & 3] * w2[(corner >> 2) & 1]
                    acc0 = acc0 + wc * rows[pl.ds(corner * _C + o, 16)]
                    acc1 = acc1 + wc * rows[pl.ds(8 * _C + corner * _C + o, 16)]
                outb[2 * l, pl.ds(o, 16)] = acc0
                outb[2 * l + 1, pl.ds(o, 16)] = acc1
                return c2

            lax.fori_loop(0, _G, pass2, 0)

        pltpu.sync_copy(outb, out_hbm.at[:, pl.ds(gbase, _C)])
        return carry

    lax.fori_loop(0, _NCH, chunk_body, 0)


@jax.jit
def kernel(x, table):
    x0 = x[:, 0]
    x1 = x[:, 1]
    x2 = x[:, 2]
    table_lin = (
        table.reshape(_L, _H // 128, 128, _F)
        .transpose(0, 1, 3, 2)
        .reshape(_L * _H * _F)
    )
    mesh = plsc.VectorSubcoreMesh(core_axis_name="c", subcore_axis_name="s")
    run = functools.partial(
        pl.kernel,
        out_type=jax.ShapeDtypeStruct((_L * _F, _N), jnp.float32),
        mesh=mesh,
        scratch_types=[
            pltpu.VMEM((_C,), jnp.float32),
            pltpu.VMEM((_C,), jnp.float32),
            pltpu.VMEM((_C,), jnp.float32),
            pltpu.VMEM((16 * _C,), jnp.int32),
            pltpu.VMEM((16 * _C,), jnp.float32),
            pltpu.VMEM((_L * _F, _C), jnp.float32),
            pltpu.SemaphoreType.DMA,
        ],
        compiler_params=pltpu.CompilerParams(
            needs_layout_passes=False, use_tc_tiling_on_sc=False),
    )(_sc_kernel)
    out = run(x0, x1, x2, table_lin)
    return out.T
